# baseline (speedup 1.0000x reference)
import functools

import jax
import jax.numpy as jnp
from jax import lax
from jax.experimental import pallas as pl
from jax.experimental.pallas import tpu as pltpu
from jax.experimental.pallas import tpu_sc as plsc

BATCH = 16384
FACTORS = 64
NUM_WORKERS = 32
ROWS_PER_W = BATCH // NUM_WORKERS
CHUNK = 128
NCHUNK = ROWS_PER_W // CHUNK
GROUPS = ROWS_PER_W // 16

_mesh = plsc.VectorSubcoreMesh(core_axis_name="c", subcore_axis_name="s")


@functools.partial(
    pl.kernel,
    mesh=_mesh,
    out_type=jax.ShapeDtypeStruct((BATCH,), jnp.float32),
    scratch_types=[
        pltpu.VMEM((NCHUNK, CHUNK), jnp.int32),
        pltpu.VMEM((NCHUNK, CHUNK), jnp.int32),
        pltpu.VMEM((ROWS_PER_W, FACTORS), jnp.float32),
        pltpu.VMEM((ROWS_PER_W, FACTORS), jnp.float32),
        pltpu.VMEM((FACTORS,), jnp.float32),
        pltpu.VMEM((16,), jnp.float32),
        pltpu.VMEM((ROWS_PER_W,), jnp.float32),
        pltpu.SemaphoreType.DMA,
    ],
    compiler_params=pltpu.CompilerParams(
        needs_layout_passes=False, use_tc_tiling_on_sc=False),
)
def _mf_sc(uidx_hbm, iidx_hbm, ut_hbm, it_hbm, w_hbm, b_hbm, out_hbm,
           uix_v, iix_v, ru_v, ri_v, w_v, b_v, out_v, sem):
    wid = lax.axis_index("s") * 2 + lax.axis_index("c")

    pltpu.sync_copy(uidx_hbm.at[wid], uix_v)
    pltpu.sync_copy(iidx_hbm.at[wid], iix_v)
    pltpu.sync_copy(w_hbm, w_v)
    pltpu.sync_copy(b_hbm, b_v)

    copies = []
    for j in range(NCHUNK):
        copies.append(pltpu.async_copy(
            ut_hbm.at[uix_v.at[j]], ru_v.at[pl.ds(j * CHUNK, CHUNK)], sem))
        copies.append(pltpu.async_copy(
            it_hbm.at[iix_v.at[j]], ri_v.at[pl.ds(j * CHUNK, CHUNK)], sem))
    for c in copies:
        c.wait()

    iota16 = lax.iota(jnp.int32, 16)
    last_lane = iota16 == 15
    b_onehot = jnp.where(iota16 == 0, b_v[...], 0.0)
    wv = [w_v[pl.ds(k * 16, 16)] for k in range(FACTORS // 16)]

    UNROLL = 8

    def row_block(r0, carry):
        for s in range(UNROLL):
            r = r0 * UNROLL + s
            acc = b_onehot
            for k in range(FACTORS // 16):
                u = ru_v[r, pl.ds(k * 16, 16)]
                v = ri_v[r, pl.ds(k * 16, 16)]
                acc = acc + u * v * wv[k]
            tot = plsc.cumsum(acc)
            plsc.store_scatter(out_v, [jnp.full((16,), r, jnp.int32)],
                               tot, mask=last_lane)
        return carry

    lax.fori_loop(0, ROWS_PER_W // UNROLL, row_block, 0)

    pltpu.sync_copy(out_v, out_hbm.at[pl.ds(wid * ROWS_PER_W, ROWS_PER_W)])


def kernel(user_idx, item_idx, user_table, item_table, W, b):
    uidx = user_idx.reshape(NUM_WORKERS, NCHUNK, CHUNK)
    iidx = item_idx.reshape(NUM_WORKERS, NCHUNK, CHUNK)
    w = W.reshape(FACTORS)
    bvec = jnp.broadcast_to(b, (16,)).astype(jnp.float32)
    return _mf_sc(uidx, iidx, user_table, item_table, w, bvec)
